# Initial kernel scaffold; baseline (speedup 1.0000x reference)
import jax
import jax.numpy as jnp
from jax.experimental import pallas as pl


def kernel(attention_weights, edge_types, E_v):
    raise NotImplementedError("write your pallas kernel here")


# speedup vs baseline: 119.8856x; 119.8856x over previous
import functools

import jax
import jax.numpy as jnp
from jax import lax
from jax.experimental import pallas as pl

_C = 24
_DH = 64


def _tc_body(aw_ref, et_ref, ev_ref, out_ref):
    aw = aw_ref[0]
    et = et_ref[0]
    iota_c = lax.broadcasted_iota(jnp.int32, et.shape + (_C,), 2)
    onehot = (et[:, :, None] == iota_c).astype(jnp.float32)
    supp = lax.dot_general(
        aw, onehot,
        dimension_numbers=(((2,), (1,)), ((1,), (0,))),
        preferred_element_type=jnp.float32,
    )
    supp_t = jnp.transpose(supp, (1, 0, 2))
    out = lax.dot_general(
        supp_t, ev_ref[...],
        dimension_numbers=(((2,), (1,)), ((0,), (0,))),
        preferred_element_type=jnp.float32,
    )
    out_ref[0] = out


@jax.jit
def kernel(attention_weights, edge_types, E_v):
    b, h, n, _ = attention_weights.shape
    c = E_v.shape[0]
    et32 = edge_types.astype(jnp.int32)
    ev_h = jnp.transpose(E_v.reshape(c, h, _DH), (1, 0, 2))

    R = 64
    grid = (b, n // R)
    out = pl.pallas_call(
        _tc_body,
        grid=grid,
        in_specs=[
            pl.BlockSpec((1, h, R, n), lambda bi, ri: (bi, 0, ri, 0)),
            pl.BlockSpec((1, R, n), lambda bi, ri: (bi, ri, 0)),
            pl.BlockSpec((h, c, _DH), lambda bi, ri: (0, 0, 0)),
        ],
        out_specs=pl.BlockSpec((1, h, R, _DH), lambda bi, ri: (bi, 0, ri, 0)),
        out_shape=jax.ShapeDtypeStruct((b, h, n, _DH), jnp.float32),
    )(attention_weights, et32, ev_h)
    return out
